# Initial kernel scaffold; baseline (speedup 1.0000x reference)
import jax
import jax.numpy as jnp
from jax import lax
from jax.experimental import pallas as pl
from jax.experimental.pallas import tpu as pltpu
from jax.experimental.pallas import tpu_sc as plsc

N = 10000
E = 160000
D = 256
HALF = 128
NC = 2
NS = 16
LANES = 16
CHUNK = 128
NCHUNKS = E // CHUNK
ROWS_PER_TILE = N // NS
BLK = 1000


def _mm_body(x_ref, w_ref, b_ref, out_ref):
    acc = jnp.dot(x_ref[...], w_ref[...], preferred_element_type=jnp.float32)
    out_ref[...] = acc + b_ref[0][None, :]


def _support1(x, W1, b1_2h):
    grid = (NC, N // BLK)
    return pl.pallas_call(
        _mm_body,
        grid=grid,
        in_specs=[
            pl.BlockSpec((BLK, D), lambda i, j: (j, 0)),
            pl.BlockSpec((D, HALF), lambda i, j: (0, i)),
            pl.BlockSpec((1, HALF), lambda i, j: (i, 0)),
        ],
        out_specs=pl.BlockSpec((BLK, HALF), lambda i, j: (i * (N // BLK) + j, 0)),
        out_shape=jax.ShapeDtypeStruct((NC * N, HALF), jnp.float32),
    )(x, W1, b1_2h)


def _norm_mm_body(a0_ref, a1_ref, d0_ref, d1_ref, w_ref, b_ref, out_ref):
    deg = d0_ref[:, 0:1] + d1_ref[:, 0:1]
    inv = 1.0 / jnp.maximum(deg, 1.0)
    h0 = jnp.maximum(a0_ref[...] * inv, 0.0)
    h1 = jnp.maximum(a1_ref[...] * inv, 0.0)
    acc = jnp.dot(h0, w_ref[0:HALF, :], preferred_element_type=jnp.float32)
    acc += jnp.dot(h1, w_ref[HALF:D, :], preferred_element_type=jnp.float32)
    out_ref[...] = acc + b_ref[0][None, :]


def _support2(agg1, deg, W2, b2_2h):
    nb = N // BLK
    grid = (NC, nb)
    return pl.pallas_call(
        _norm_mm_body,
        grid=grid,
        in_specs=[
            pl.BlockSpec((BLK, HALF), lambda i, j: (j, 0)),
            pl.BlockSpec((BLK, HALF), lambda i, j: (nb + j, 0)),
            pl.BlockSpec((BLK, LANES), lambda i, j: (j, 0)),
            pl.BlockSpec((BLK, LANES), lambda i, j: (nb + j, 0)),
            pl.BlockSpec((D, HALF), lambda i, j: (0, i)),
            pl.BlockSpec((1, HALF), lambda i, j: (i, 0)),
        ],
        out_specs=pl.BlockSpec((BLK, HALF), lambda i, j: (i * nb + j, 0)),
        out_shape=jax.ShapeDtypeStruct((NC * N, HALF), jnp.float32),
    )(agg1, agg1, deg, deg, W2, b2_2h)


def _final_body(a0_ref, a1_ref, d0_ref, d1_ref, out_ref):
    deg = d0_ref[:, 0:1] + d1_ref[:, 0:1]
    inv = 1.0 / jnp.maximum(deg, 1.0)
    out_ref[:, 0:HALF] = a0_ref[...] * inv
    out_ref[:, HALF:D] = a1_ref[...] * inv


def _finalize(agg2, deg):
    nb = N // BLK
    return pl.pallas_call(
        _final_body,
        grid=(nb,),
        in_specs=[
            pl.BlockSpec((BLK, HALF), lambda j: (j, 0)),
            pl.BlockSpec((BLK, HALF), lambda j: (nb + j, 0)),
            pl.BlockSpec((BLK, LANES), lambda j: (j, 0)),
            pl.BlockSpec((BLK, LANES), lambda j: (nb + j, 0)),
        ],
        out_specs=pl.BlockSpec((BLK, D), lambda j: (j, 0)),
        out_shape=jax.ShapeDtypeStruct((N, D), jnp.float32),
    )(agg2, agg2, deg, deg)


def _sc_body_common(with_deg, sup_hbm, src_hbm, dst_hbm, zeros_hbm, zeros16_hbm,
                    ones_hbm, agg_out, deg_out, src_v, dst_v, rows_v, ones_v,
                    acc_sh, deg_sh, sem):
    cid = lax.axis_index("c")
    sid = lax.axis_index("s")

    rbase = sid * ROWS_PER_TILE
    pltpu.sync_copy(zeros_hbm.at[pl.ds(rbase, ROWS_PER_TILE)],
                    acc_sh.at[pl.ds(rbase, ROWS_PER_TILE)])
    if with_deg:
        pltpu.sync_copy(zeros16_hbm.at[pl.ds(rbase, ROWS_PER_TILE)],
                        deg_sh.at[pl.ds(rbase, ROWS_PER_TILE)])
        pltpu.sync_copy(ones_hbm, ones_v)
    plsc.subcore_barrier()

    lo = sid * NCHUNKS // NS
    hi = (sid + 1) * NCHUNKS // NS
    dlo = cid * (NCHUNKS // NC)
    dhi = dlo + NCHUNKS // NC
    off = cid * N

    def chunk_step(k, carry):
        base = k * CHUNK
        pltpu.sync_copy(src_hbm.at[pl.ds(base, CHUNK)], src_v)
        pltpu.sync_copy(dst_hbm.at[pl.ds(base, CHUNK)], dst_v)
        offv = jnp.full((LANES,), off, jnp.int32)
        for i in range(CHUNK // LANES):
            sl = pl.ds(i * LANES, LANES)
            src_v[sl] = src_v[sl] + offv
        pltpu.async_copy(sup_hbm.at[src_v], rows_v, sem).wait()
        pltpu.sync_copy(rows_v, acc_sh.at[dst_v], add=True)
        if with_deg:
            @pl.when(jnp.logical_and(k >= dlo, k < dhi))
            def _():
                pltpu.sync_copy(ones_v, deg_sh.at[dst_v], add=True)
        return carry

    lax.fori_loop(lo, hi, chunk_step, 0)
    plsc.subcore_barrier()

    obase = cid * N + rbase
    pltpu.sync_copy(acc_sh.at[pl.ds(rbase, ROWS_PER_TILE)],
                    agg_out.at[pl.ds(obase, ROWS_PER_TILE)])
    if with_deg:
        pltpu.sync_copy(deg_sh.at[pl.ds(rbase, ROWS_PER_TILE)],
                        deg_out.at[pl.ds(obase, ROWS_PER_TILE)])


def _make_sc_kernel(with_deg):
    mesh = plsc.VectorSubcoreMesh(core_axis_name="c", subcore_axis_name="s")
    if with_deg:
        out_type = (jax.ShapeDtypeStruct((NC * N, HALF), jnp.float32),
                    jax.ShapeDtypeStruct((NC * N, LANES), jnp.float32))
    else:
        out_type = jax.ShapeDtypeStruct((NC * N, HALF), jnp.float32)
    scratch = [
        pltpu.VMEM((CHUNK,), jnp.int32),
        pltpu.VMEM((CHUNK,), jnp.int32),
        pltpu.VMEM((CHUNK, HALF), jnp.float32),
        pltpu.VMEM((CHUNK, LANES), jnp.float32),
        pltpu.VMEM_SHARED((N, HALF), jnp.float32),
        pltpu.VMEM_SHARED((N, LANES), jnp.float32),
        pltpu.SemaphoreType.DMA,
    ]

    if with_deg:
        def body(sup, src, dst, z, z16, ones, agg_out, deg_out,
                 src_v, dst_v, rows_v, ones_v, acc_sh, deg_sh, sem):
            _sc_body_common(True, sup, src, dst, z, z16, ones, agg_out, deg_out,
                            src_v, dst_v, rows_v, ones_v, acc_sh, deg_sh, sem)
    else:
        def body(sup, src, dst, z, z16, ones, agg_out,
                 src_v, dst_v, rows_v, ones_v, acc_sh, deg_sh, sem):
            _sc_body_common(False, sup, src, dst, z, z16, ones, agg_out, None,
                            src_v, dst_v, rows_v, ones_v, acc_sh, deg_sh, sem)

    return pl.kernel(body, out_type=out_type, mesh=mesh, scratch_types=scratch)


_sc_layer1 = _make_sc_kernel(True)
_sc_layer2 = _make_sc_kernel(False)


def kernel(x, edge_index, W1, b1, W2, b2):
    src = edge_index[0]
    dst = edge_index[1]
    b1_2h = b1.reshape(NC, HALF)
    b2_2h = b2.reshape(NC, HALF)
    zeros = jnp.zeros((N, HALF), jnp.float32)
    zeros16 = jnp.zeros((N, LANES), jnp.float32)
    ones = jnp.ones((CHUNK, LANES), jnp.float32)

    sup1 = _support1(x, W1, b1_2h)
    agg1, deg = _sc_layer1(sup1, src, dst, zeros, zeros16, ones)
    sup2 = _support2(agg1, deg, W2, b2_2h)
    agg2 = _sc_layer2(sup2, src, dst, zeros, zeros16, ones)
    return _finalize(agg2, deg)


# speedup vs baseline: 3.3298x; 3.3298x over previous
import jax
import jax.numpy as jnp
from jax import lax
from jax.experimental import pallas as pl
from jax.experimental.pallas import tpu as pltpu
from jax.experimental.pallas import tpu_sc as plsc

N = 10000
E = 160000
D = 256
HALF = 128
NC = 2
NS = 16
CHUNK = 80
EDGES_PER_TILE = E // NS
NCHUNKS = EDGES_PER_TILE // CHUNK
DCHUNK = 40
DEG_PER_WORKER = E // (NC * NS)
DNCHUNKS = DEG_PER_WORKER // DCHUNK
SPAN = 624
TAIL = N - SPAN * NS
BLK = 1000
NB = N // BLK


def _mm_body(x_ref, w_ref, b_ref, out_ref):
    acc = jnp.dot(x_ref[...], w_ref[...], preferred_element_type=jnp.float32)
    out_ref[...] = acc + b_ref[0]


def _support1(x, W1, b1_2h):
    return pl.pallas_call(
        _mm_body,
        grid=(NC, NB),
        in_specs=[
            pl.BlockSpec((BLK, D), lambda i, j: (j, 0)),
            pl.BlockSpec((D, HALF), lambda i, j: (0, i)),
            pl.BlockSpec((1, 1, HALF), lambda i, j: (i, 0, 0)),
        ],
        out_specs=pl.BlockSpec((BLK, HALF), lambda i, j: (i * NB + j, 0)),
        out_shape=jax.ShapeDtypeStruct((NC * N, HALF), jnp.float32),
    )(x, W1, b1_2h)


def _norm_mm_body(a0_ref, a1_ref, d0_ref, d1_ref, w_ref, b_ref, out_ref):
    deg = d0_ref[:, 0:1] + d1_ref[:, 0:1]
    inv = 1.0 / jnp.maximum(deg, 1.0)
    h0 = jnp.maximum(a0_ref[...] * inv, 0.0)
    h1 = jnp.maximum(a1_ref[...] * inv, 0.0)
    acc = jnp.dot(h0, w_ref[0:HALF, :], preferred_element_type=jnp.float32)
    acc += jnp.dot(h1, w_ref[HALF:D, :], preferred_element_type=jnp.float32)
    out_ref[...] = acc + b_ref[0]


def _support2(agg1, deg, W2, b2_2h):
    return pl.pallas_call(
        _norm_mm_body,
        grid=(NC, NB),
        in_specs=[
            pl.BlockSpec((BLK, HALF), lambda i, j: (j, 0)),
            pl.BlockSpec((BLK, HALF), lambda i, j: (NB + j, 0)),
            pl.BlockSpec((BLK, HALF), lambda i, j: (j, 0)),
            pl.BlockSpec((BLK, HALF), lambda i, j: (NB + j, 0)),
            pl.BlockSpec((D, HALF), lambda i, j: (0, i)),
            pl.BlockSpec((1, 1, HALF), lambda i, j: (i, 0, 0)),
        ],
        out_specs=pl.BlockSpec((BLK, HALF), lambda i, j: (i * NB + j, 0)),
        out_shape=jax.ShapeDtypeStruct((NC * N, HALF), jnp.float32),
    )(agg1, agg1, deg, deg, W2, b2_2h)


def _final_body(a0_ref, a1_ref, d0_ref, d1_ref, out_ref):
    deg = d0_ref[:, 0:1] + d1_ref[:, 0:1]
    inv = 1.0 / jnp.maximum(deg, 1.0)
    out_ref[:, 0:HALF] = a0_ref[...] * inv
    out_ref[:, HALF:D] = a1_ref[...] * inv


def _finalize(agg2, deg):
    return pl.pallas_call(
        _final_body,
        grid=(NB,),
        in_specs=[
            pl.BlockSpec((BLK, HALF), lambda j: (j, 0)),
            pl.BlockSpec((BLK, HALF), lambda j: (NB + j, 0)),
            pl.BlockSpec((BLK, HALF), lambda j: (j, 0)),
            pl.BlockSpec((BLK, HALF), lambda j: (NB + j, 0)),
        ],
        out_specs=pl.BlockSpec((BLK, D), lambda j: (j, 0)),
        out_shape=jax.ShapeDtypeStruct((N, D), jnp.float32),
    )(agg2, agg2, deg, deg)


def _zero_acc(zeros_hbm, acc_sh, sid):
    rbase = sid * SPAN
    pltpu.sync_copy(zeros_hbm.at[pl.ds(rbase, SPAN)],
                    acc_sh.at[pl.ds(rbase, SPAN)])

    @pl.when(sid == 0)
    def _():
        pltpu.sync_copy(zeros_hbm.at[pl.ds(SPAN * NS, TAIL)],
                        acc_sh.at[pl.ds(SPAN * NS, TAIL)])


def _write_acc(acc_sh, out_hbm, cid, sid):
    rbase = sid * SPAN
    pltpu.sync_copy(acc_sh.at[pl.ds(rbase, SPAN)],
                    out_hbm.at[pl.ds(cid * N + rbase, SPAN)])

    @pl.when(sid == 0)
    def _():
        pltpu.sync_copy(acc_sh.at[pl.ds(SPAN * NS, TAIL)],
                        out_hbm.at[pl.ds(cid * N + SPAN * NS, TAIL)])


def _sc_body_common(with_deg, sup_hbm, srcall_hbm, dst_hbm, zeros_hbm,
                    ones_hbm, agg_out, deg_out,
                    src_v, dst_v, dstd_v, rows_v, ones_v, acc_sh):
    cid = lax.axis_index("c")
    sid = lax.axis_index("s")

    _zero_acc(zeros_hbm, acc_sh, sid)
    if with_deg:
        pltpu.sync_copy(ones_hbm, ones_v)
    plsc.subcore_barrier()

    if with_deg:
        wbase = (cid * NS + sid) * DEG_PER_WORKER

        def deg_step(k, carry):
            pltpu.sync_copy(dst_hbm.at[pl.ds(wbase + k * DCHUNK, DCHUNK)],
                            dstd_v)
            pltpu.sync_copy(ones_v, acc_sh.at[dstd_v], add=True)
            return carry

        lax.fori_loop(0, DNCHUNKS, deg_step, 0)
        plsc.subcore_barrier()
        _write_acc(acc_sh, deg_out, cid, sid)
        _zero_acc(zeros_hbm, acc_sh, sid)
        plsc.subcore_barrier()

    ebase = cid * E + sid * EDGES_PER_TILE
    dbase = sid * EDGES_PER_TILE

    def chunk_step(k, carry):
        off = k * CHUNK
        pltpu.sync_copy(srcall_hbm.at[pl.ds(ebase + off, CHUNK)], src_v)
        pltpu.sync_copy(dst_hbm.at[pl.ds(dbase + off, CHUNK)], dst_v)
        pltpu.sync_copy(sup_hbm.at[src_v], rows_v)
        pltpu.sync_copy(rows_v, acc_sh.at[dst_v], add=True)
        return carry

    lax.fori_loop(0, NCHUNKS, chunk_step, 0)
    plsc.subcore_barrier()
    _write_acc(acc_sh, agg_out, cid, sid)


def _make_sc_kernel(with_deg):
    mesh = plsc.VectorSubcoreMesh(core_axis_name="c", subcore_axis_name="s")
    if with_deg:
        out_type = (jax.ShapeDtypeStruct((NC * N, HALF), jnp.float32),
                    jax.ShapeDtypeStruct((NC * N, HALF), jnp.float32))
    else:
        out_type = jax.ShapeDtypeStruct((NC * N, HALF), jnp.float32)
    scratch = [
        pltpu.VMEM((CHUNK,), jnp.int32),
        pltpu.VMEM((CHUNK,), jnp.int32),
        pltpu.VMEM((DCHUNK,), jnp.int32),
        pltpu.VMEM((CHUNK, HALF), jnp.float32),
        pltpu.VMEM((DCHUNK, HALF), jnp.float32),
        pltpu.VMEM_SHARED((N, HALF), jnp.float32),
    ]

    if with_deg:
        def body(sup, srcall, dst, z, ones, agg_out, deg_out,
                 src_v, dst_v, dstd_v, rows_v, ones_v, acc_sh):
            _sc_body_common(True, sup, srcall, dst, z, ones,
                            agg_out, deg_out,
                            src_v, dst_v, dstd_v, rows_v, ones_v, acc_sh)
    else:
        def body(sup, srcall, dst, z, ones, agg_out,
                 src_v, dst_v, dstd_v, rows_v, ones_v, acc_sh):
            _sc_body_common(False, sup, srcall, dst, z, ones,
                            agg_out, None,
                            src_v, dst_v, dstd_v, rows_v, ones_v, acc_sh)

    return pl.kernel(body, out_type=out_type, mesh=mesh, scratch_types=scratch)


_sc_layer1 = _make_sc_kernel(True)
_sc_layer2 = _make_sc_kernel(False)


def kernel(x, edge_index, W1, b1, W2, b2):
    src = edge_index[0]
    dst = edge_index[1]
    src_all = jnp.concatenate([src, src + N])
    b1_2h = b1.reshape(NC, 1, HALF)
    b2_2h = b2.reshape(NC, 1, HALF)
    zeros = jnp.zeros((N, HALF), jnp.float32)
    ones = jnp.ones((DCHUNK, HALF), jnp.float32)

    sup1 = _support1(x, W1, b1_2h)
    agg1, deg = _sc_layer1(sup1, src_all, dst, zeros, ones)
    sup2 = _support2(agg1, deg, W2, b2_2h)
    agg2 = _sc_layer2(sup2, src_all, dst, zeros, ones)
    return _finalize(agg2, deg)


# speedup vs baseline: 3.4950x; 1.0496x over previous
import jax
import jax.numpy as jnp
from jax import lax
from jax.experimental import pallas as pl
from jax.experimental.pallas import tpu as pltpu
from jax.experimental.pallas import tpu_sc as plsc

N = 10000
E = 160000
D = 256
HALF = 128
NC = 2
NS = 16
CW = 128
EDGES_PER_TILE = E // NS
NCH = 80
DEG_PER_WORKER = E // (NC * NS)
DNCH = 40
NROWS = N + 8
SPAN = 624
TAIL = N - SPAN * NS
BLK = 1000
NB = N // BLK


def _mm_body(x_ref, w_ref, b_ref, out_ref):
    acc = jnp.dot(x_ref[...], w_ref[...], preferred_element_type=jnp.float32)
    out_ref[...] = acc + b_ref[0]


def _support1(x, W1, b1_2h):
    return pl.pallas_call(
        _mm_body,
        grid=(NC, NB),
        in_specs=[
            pl.BlockSpec((BLK, D), lambda i, j: (j, 0)),
            pl.BlockSpec((D, HALF), lambda i, j: (0, i)),
            pl.BlockSpec((1, 1, HALF), lambda i, j: (i, 0, 0)),
        ],
        out_specs=pl.BlockSpec((BLK, HALF), lambda i, j: (i * NB + j, 0)),
        out_shape=jax.ShapeDtypeStruct((NC * N, HALF), jnp.float32),
    )(x, W1, b1_2h)


def _norm_mm_body(a0_ref, a1_ref, d0_ref, d1_ref, w_ref, b_ref, out_ref):
    deg = d0_ref[:, 0:1] + d1_ref[:, 0:1]
    inv = 1.0 / jnp.maximum(deg, 1.0)
    h0 = jnp.maximum(a0_ref[...] * inv, 0.0)
    h1 = jnp.maximum(a1_ref[...] * inv, 0.0)
    acc = jnp.dot(h0, w_ref[0:HALF, :], preferred_element_type=jnp.float32)
    acc += jnp.dot(h1, w_ref[HALF:D, :], preferred_element_type=jnp.float32)
    out_ref[...] = acc + b_ref[0]


def _support2(agg1, deg, W2, b2_2h):
    return pl.pallas_call(
        _norm_mm_body,
        grid=(NC, NB),
        in_specs=[
            pl.BlockSpec((BLK, HALF), lambda i, j: (j, 0)),
            pl.BlockSpec((BLK, HALF), lambda i, j: (NB + j, 0)),
            pl.BlockSpec((BLK, HALF), lambda i, j: (j, 0)),
            pl.BlockSpec((BLK, HALF), lambda i, j: (NB + j, 0)),
            pl.BlockSpec((D, HALF), lambda i, j: (0, i)),
            pl.BlockSpec((1, 1, HALF), lambda i, j: (i, 0, 0)),
        ],
        out_specs=pl.BlockSpec((BLK, HALF), lambda i, j: (i * NB + j, 0)),
        out_shape=jax.ShapeDtypeStruct((NC * N, HALF), jnp.float32),
    )(agg1, agg1, deg, deg, W2, b2_2h)


def _final_body(a0_ref, a1_ref, d0_ref, d1_ref, out_ref):
    deg = d0_ref[:, 0:1] + d1_ref[:, 0:1]
    inv = 1.0 / jnp.maximum(deg, 1.0)
    out_ref[:, 0:HALF] = a0_ref[...] * inv
    out_ref[:, HALF:D] = a1_ref[...] * inv


def _finalize(agg2, deg):
    return pl.pallas_call(
        _final_body,
        grid=(NB,),
        in_specs=[
            pl.BlockSpec((BLK, HALF), lambda j: (j, 0)),
            pl.BlockSpec((BLK, HALF), lambda j: (NB + j, 0)),
            pl.BlockSpec((BLK, HALF), lambda j: (j, 0)),
            pl.BlockSpec((BLK, HALF), lambda j: (NB + j, 0)),
        ],
        out_specs=pl.BlockSpec((BLK, D), lambda j: (j, 0)),
        out_shape=jax.ShapeDtypeStruct((N, D), jnp.float32),
    )(agg2, agg2, deg, deg)


def _zero_acc(zeros_hbm, acc_sh, sid):
    rbase = sid * SPAN
    pltpu.sync_copy(zeros_hbm.at[pl.ds(rbase, SPAN)],
                    acc_sh.at[pl.ds(rbase, SPAN)])

    @pl.when(sid == 0)
    def _():
        pltpu.sync_copy(zeros_hbm.at[pl.ds(SPAN * NS, NROWS - SPAN * NS)],
                        acc_sh.at[pl.ds(SPAN * NS, NROWS - SPAN * NS)])


def _write_acc(acc_sh, out_hbm, cid, sid):
    rbase = sid * SPAN
    pltpu.sync_copy(acc_sh.at[pl.ds(rbase, SPAN)],
                    out_hbm.at[pl.ds(cid * N + rbase, SPAN)])

    @pl.when(sid == 0)
    def _():
        pltpu.sync_copy(acc_sh.at[pl.ds(SPAN * NS, TAIL)],
                        out_hbm.at[pl.ds(cid * N + SPAN * NS, TAIL)])


def _sc_body_common(with_deg, sup_hbm, src3_hbm, dst3_hbm, dstd3_hbm,
                    zeros_hbm, ones_hbm, agg_out, deg_out,
                    src_v, dst_v, rows_v, acc_sh):
    cid = lax.axis_index("c")
    sid = lax.axis_index("s")

    pltpu.sync_copy(dst3_hbm.at[pl.ds(sid * NCH, NCH)], dst_v)
    if with_deg:
        pltpu.sync_copy(dstd3_hbm.at[pl.ds((cid * NS + sid) * DNCH, DNCH)],
                        src_v.at[pl.ds(0, DNCH)])
        pltpu.sync_copy(ones_hbm, rows_v)
    _zero_acc(zeros_hbm, acc_sh, sid)
    plsc.subcore_barrier()

    if with_deg:
        def deg_step(k, carry):
            pltpu.sync_copy(rows_v, acc_sh.at[src_v.at[k]], add=True)
            return carry

        lax.fori_loop(0, DNCH, deg_step, 0)
        plsc.subcore_barrier()
        _write_acc(acc_sh, deg_out, cid, sid)
        _zero_acc(zeros_hbm, acc_sh, sid)
        plsc.subcore_barrier()

    pltpu.sync_copy(src3_hbm.at[pl.ds((cid * NS + sid) * NCH, NCH)], src_v)

    def chunk_step(k, carry):
        pltpu.sync_copy(sup_hbm.at[src_v.at[k]], rows_v)
        pltpu.sync_copy(rows_v, acc_sh.at[dst_v.at[k]], add=True)
        return carry

    lax.fori_loop(0, NCH, chunk_step, 0)
    plsc.subcore_barrier()
    _write_acc(acc_sh, agg_out, cid, sid)


def _make_sc_kernel(with_deg):
    mesh = plsc.VectorSubcoreMesh(core_axis_name="c", subcore_axis_name="s")
    if with_deg:
        out_type = (jax.ShapeDtypeStruct((NC * N, HALF), jnp.float32),
                    jax.ShapeDtypeStruct((NC * N, HALF), jnp.float32))
    else:
        out_type = jax.ShapeDtypeStruct((NC * N, HALF), jnp.float32)
    scratch = [
        pltpu.VMEM((NCH, CW), jnp.int32),
        pltpu.VMEM((NCH, CW), jnp.int32),
        pltpu.VMEM((CW, HALF), jnp.float32),
        pltpu.VMEM_SHARED((NROWS, HALF), jnp.float32),
    ]

    if with_deg:
        def body(sup, src3, dst3, dstd3, z, ones, agg_out, deg_out,
                 src_v, dst_v, rows_v, acc_sh):
            _sc_body_common(True, sup, src3, dst3, dstd3, z, ones,
                            agg_out, deg_out,
                            src_v, dst_v, rows_v, acc_sh)
    else:
        def body(sup, src3, dst3, dstd3, z, ones, agg_out,
                 src_v, dst_v, rows_v, acc_sh):
            _sc_body_common(False, sup, src3, dst3, dstd3, z, ones,
                            agg_out, None,
                            src_v, dst_v, rows_v, acc_sh)

    return pl.kernel(body, out_type=out_type, mesh=mesh, scratch_types=scratch)


_sc_layer1 = _make_sc_kernel(True)
_sc_layer2 = _make_sc_kernel(False)


def kernel(x, edge_index, W1, b1, W2, b2):
    src = edge_index[0]
    dst = edge_index[1]
    padw = NCH * CW - EDGES_PER_TILE
    sp = jnp.concatenate(
        [src.reshape(NS, EDGES_PER_TILE),
         jnp.zeros((NS, padw), jnp.int32)], axis=1)
    src3 = jnp.concatenate([sp, sp + N], axis=0).reshape(NC * NS * NCH, CW)
    dst3 = jnp.concatenate(
        [dst.reshape(NS, EDGES_PER_TILE),
         jnp.full((NS, padw), N, jnp.int32)], axis=1).reshape(NS * NCH, CW)
    dpadw = DNCH * CW - DEG_PER_WORKER
    dstd3 = jnp.concatenate(
        [dst.reshape(NC * NS, DEG_PER_WORKER),
         jnp.full((NC * NS, dpadw), N, jnp.int32)],
        axis=1).reshape(NC * NS * DNCH, CW)
    b1_2h = b1.reshape(NC, 1, HALF)
    b2_2h = b2.reshape(NC, 1, HALF)
    zeros = jnp.zeros((NROWS, HALF), jnp.float32)
    ones = jnp.ones((CW, HALF), jnp.float32)

    sup1 = _support1(x, W1, b1_2h)
    agg1, deg = _sc_layer1(sup1, src3, dst3, dstd3, zeros, ones)
    sup2 = _support2(agg1, deg, W2, b2_2h)
    agg2 = _sc_layer2(sup2, src3, dst3, dstd3, zeros, ones)
    return _finalize(agg2, deg)
